# Initial kernel scaffold; baseline (speedup 1.0000x reference)
import jax
import jax.numpy as jnp
from jax.experimental import pallas as pl


def kernel(f_input, topic_vector, rec_vector_1, rec_vector_2, clf_w, clf_b):
    raise NotImplementedError("write your pallas kernel here")


# speedup vs baseline: 5.3459x; 5.3459x over previous
import functools

import jax
import jax.numpy as jnp
from jax import lax
from jax.experimental import pallas as pl
from jax.experimental.pallas import tpu as pltpu

THRES = 0.3
EPS_NORM = 1e-12


def _fwd_kernel(f_ref, tvn_ref, r1_ref, r2_ref, wf_ref, bc_ref,
                pred_ref, tpnt_ref, tpnn_ref, ae_ref):
    f = f_ref[...]

    sumsq = jnp.sum(f * f, axis=-1, keepdims=True)
    inv_norm = lax.rsqrt(jnp.maximum(sumsq, EPS_NORM * EPS_NORM))

    topic_prob = jnp.dot(f, tvn_ref[...], preferred_element_type=jnp.float32)
    topic_prob_n = topic_prob * inv_norm

    mask = (topic_prob_n > THRES).astype(jnp.float32)
    topic_prob_am = topic_prob * mask
    topic_prob_sum = jnp.sum(topic_prob_am, axis=-1, keepdims=True) + 0.001
    topic_prob_nn = topic_prob_am * pl.reciprocal(topic_prob_sum, approx=True)

    rec1 = jnp.maximum(
        jnp.dot(topic_prob_nn, r1_ref[...], preferred_element_type=jnp.float32), 0.0)
    rec2 = jnp.dot(rec1, r2_ref[...], preferred_element_type=jnp.float32)
    pred = jnp.dot(rec1, wf_ref[...], preferred_element_type=jnp.float32) + bc_ref[...]

    fn = f * inv_norm
    diff = fn - rec2
    tile_ae = jnp.sum(diff * diff)

    pred_ref[...] = pred
    tpnt_ref[...] = topic_prob_n.T
    tpnn_ref[...] = topic_prob_nn
    ae_ref[...] = jnp.full(ae_ref.shape, tile_ae, dtype=jnp.float32)


def _f32_keys(x):
    i = pltpu.bitcast(x, jnp.int32)
    return jnp.where(i < 0, i ^ jnp.int32(0x7FFFFFFF), i)


def _stats_kernel(k_topk, tpnt_ref, ae_ref, tvn_ref, out_ref):
    x = tpnt_ref[...]
    keys = _f32_keys(x)

    lo = jnp.min(keys, axis=1, keepdims=True) - 1
    hi = jnp.max(keys, axis=1, keepdims=True)

    def body(_, carry):
        lo, hi = carry
        mid = hi - (hi - lo) // 2
        cnt = jnp.sum((keys >= mid).astype(jnp.int32), axis=1, keepdims=True)
        ge = cnt >= k_topk
        return jnp.where(ge, mid, lo), jnp.where(ge, hi, mid - 1)

    t_key, _ = lax.fori_loop(0, 32, body, (lo, hi))
    ti = jnp.where(t_key < 0, t_key ^ jnp.int32(0x7FFFFFFF), t_key)
    t = pltpu.bitcast(ti, jnp.float32)

    gt = keys > t_key
    cnt_gt = jnp.sum(gt.astype(jnp.int32), axis=1, keepdims=True)
    sum_gt = jnp.sum(jnp.where(gt, x, 0.0), axis=1, keepdims=True)
    topk_sum = sum_gt + (k_topk - cnt_gt).astype(jnp.float32) * t
    topk_total = jnp.sum(topk_sum)

    ae_total = jnp.sum(ae_ref[...])
    s = jnp.sum(tvn_ref[...], axis=1, keepdims=True)
    far_sum = jnp.sum(s * s)

    rows = lax.broadcasted_iota(jnp.int32, (8, 128), 0)
    slab = jnp.where(rows == 0, topk_total,
                     jnp.where(rows == 1, ae_total,
                               jnp.where(rows == 2, far_sum, 0.0)))
    out_ref[...] = slab[None]


def kernel(f_input, topic_vector, rec_vector_1, rec_vector_2, clf_w, clf_b):
    B, D = f_input.shape
    C = topic_vector.shape[1]
    H = rec_vector_1.shape[1]
    K = clf_w.shape[1]

    tile_b = 512 if B % 512 == 0 else 128
    num_tiles = B // tile_b

    f_input = f_input.astype(jnp.float32)

    tvn = topic_vector / jnp.maximum(
        jnp.sqrt(jnp.sum(topic_vector * topic_vector, axis=0, keepdims=True)), EPS_NORM)
    tvn = tvn.astype(jnp.float32)
    w_fused = (rec_vector_2 @ clf_w).astype(jnp.float32)
    bc = clf_b.reshape(1, K).astype(jnp.float32)
    r1 = rec_vector_1.astype(jnp.float32)
    r2 = rec_vector_2.astype(jnp.float32)

    out_shapes = (
        jax.ShapeDtypeStruct((B, K), jnp.float32),
        jax.ShapeDtypeStruct((C, B), jnp.float32),
        jax.ShapeDtypeStruct((B, C), jnp.float32),
        jax.ShapeDtypeStruct((num_tiles * 8, 128), jnp.float32),
    )
    pred, tpnt, tpnn, ae_part = pl.pallas_call(
        _fwd_kernel,
        out_shape=out_shapes,
        grid=(num_tiles,),
        in_specs=[pl.BlockSpec((tile_b, D), lambda i: (i, 0)),
                  pl.BlockSpec((D, C), lambda i: (0, 0)),
                  pl.BlockSpec((C, H), lambda i: (0, 0)),
                  pl.BlockSpec((H, D), lambda i: (0, 0)),
                  pl.BlockSpec((H, K), lambda i: (0, 0)),
                  pl.BlockSpec((1, K), lambda i: (0, 0))],
        out_specs=[pl.BlockSpec((tile_b, K), lambda i: (i, 0)),
                   pl.BlockSpec((C, tile_b), lambda i: (0, i)),
                   pl.BlockSpec((tile_b, C), lambda i: (i, 0)),
                   pl.BlockSpec((8, 128), lambda i: (i, 0))],
        compiler_params=pltpu.CompilerParams(
            dimension_semantics=("parallel",),
            vmem_limit_bytes=64 * 1024 * 1024),
    )(f_input, tvn, r1, r2, w_fused, bc)

    k_topk = max(B // 4, 1)
    c_split = 2 if C % 2 == 0 else 1
    cb = C // c_split
    stats = pl.pallas_call(
        functools.partial(_stats_kernel, k_topk),
        out_shape=jax.ShapeDtypeStruct((c_split, 8, 128), jnp.float32),
        grid=(c_split,),
        in_specs=[pl.BlockSpec((cb, B), lambda i: (i, 0)),
                  pl.BlockSpec((num_tiles * 8, 128), lambda i: (0, 0)),
                  pl.BlockSpec((D, C), lambda i: (0, 0))],
        out_specs=pl.BlockSpec((1, 8, 128), lambda i: (i, 0, 0)),
        compiler_params=pltpu.CompilerParams(
            dimension_semantics=("parallel",),
            vmem_limit_bytes=64 * 1024 * 1024),
    )(tpnt, ae_part, tvn)

    topk_total = jnp.sum(stats[:, 0, 0])
    concept_sim = -topk_total / (C * k_topk)
    ae_loss = stats[0, 1, 0] / 1024.0 / (B * D)
    concept_far = (stats[0, 2, 0] - C) / (C * C)

    return pred, jnp.float32(0.0), concept_sim, concept_far, tpnn, ae_loss


# speedup vs baseline: 6.2512x; 1.1693x over previous
import functools

import jax
import jax.numpy as jnp
from jax import lax
from jax.experimental import pallas as pl
from jax.experimental.pallas import tpu as pltpu

THRES = 0.3
EPS_NORM = 1e-12


def _fwd_kernel(f_ref, tvn_ref, r1_ref, r2_ref, wf_ref, bc_ref,
                pred_ref, tpn16_ref, tpnn_ref, ae_ref):
    f = f_ref[...]

    sumsq = jnp.sum(f * f, axis=-1, keepdims=True)
    inv_norm = lax.rsqrt(jnp.maximum(sumsq, EPS_NORM * EPS_NORM))

    topic_prob = jnp.dot(f, tvn_ref[...], preferred_element_type=jnp.float32)
    topic_prob_n = topic_prob * inv_norm

    mask = (topic_prob_n > THRES).astype(jnp.float32)
    topic_prob_am = topic_prob * mask
    topic_prob_sum = jnp.sum(topic_prob_am, axis=-1, keepdims=True) + 0.001
    topic_prob_nn = topic_prob_am * pl.reciprocal(topic_prob_sum, approx=True)

    rec1 = jnp.maximum(
        jnp.dot(topic_prob_nn.astype(jnp.bfloat16), r1_ref[...],
                preferred_element_type=jnp.float32), 0.0)
    rec1_16 = rec1.astype(jnp.bfloat16)
    rec2 = jnp.dot(rec1_16, r2_ref[...], preferred_element_type=jnp.float32)
    pred = jnp.dot(rec1_16, wf_ref[...], preferred_element_type=jnp.float32) \
        + bc_ref[...]

    fn = f * inv_norm
    diff = fn - rec2
    tile_ae = jnp.sum(diff * diff)

    pred_ref[...] = pred
    tpn16_ref[...] = topic_prob_n.astype(jnp.bfloat16)
    tpnn_ref[...] = topic_prob_nn
    ae_ref[...] = jnp.full(ae_ref.shape, tile_ae, dtype=jnp.float32)


def _code_to_f32(u):
    p = jnp.where(u >= 0x8000, u - 0x8000, 0xFFFF - u)
    return pltpu.bitcast((p << 16).astype(jnp.int32), jnp.float32)


def _stats_kernel(k_topk, inv_ck, ae_scale, c_n,
                  tpn16_ref, ae_ref, tvn_ref, out_ref):
    x = tpn16_ref[...]

    lo = jnp.full((1, x.shape[1]), 0x3FFF, jnp.int32)
    hi = jnp.full((1, x.shape[1]), 0xC000, jnp.int32)

    def body(_, carry):
        lo, hi = carry
        mid = hi - (hi - lo) // 2
        mid_b = _code_to_f32(mid).astype(jnp.bfloat16)
        cnt = jnp.sum((x >= mid_b).astype(jnp.int32), axis=0, keepdims=True)
        ge = cnt >= k_topk
        return jnp.where(ge, mid, lo), jnp.where(ge, hi, mid - 1)

    t_code, _ = lax.fori_loop(0, 16, body, (lo, hi))
    t_f = _code_to_f32(t_code)
    t_b = t_f.astype(jnp.bfloat16)

    gt = x > t_b
    cnt_gt = jnp.sum(gt.astype(jnp.int32), axis=0, keepdims=True)
    sum_gt = jnp.sum(jnp.where(gt, x, jnp.bfloat16(0)).astype(jnp.float32),
                     axis=0, keepdims=True)
    topk_sum = sum_gt + (k_topk - cnt_gt).astype(jnp.float32) * t_f
    sim_val = -jnp.sum(topk_sum) * inv_ck

    ae_val = jnp.sum(ae_ref[...]) * ae_scale

    s = jnp.sum(tvn_ref[...], axis=1, keepdims=True)
    far_val = (jnp.sum(s * s) - c_n) * (1.0 / (c_n * c_n))

    rows = lax.broadcasted_iota(jnp.int32, (8, 128), 0)
    out_ref[...] = jnp.where(rows == 0, sim_val,
                             jnp.where(rows == 1, ae_val,
                                       jnp.where(rows == 2, far_val, 0.0)))


def kernel(f_input, topic_vector, rec_vector_1, rec_vector_2, clf_w, clf_b):
    B, D = f_input.shape
    C = topic_vector.shape[1]
    H = rec_vector_1.shape[1]
    K = clf_w.shape[1]

    tile_b = 512 if B % 512 == 0 else 128
    num_tiles = B // tile_b

    f_input = f_input.astype(jnp.float32)

    tvn = topic_vector / jnp.maximum(
        jnp.sqrt(jnp.sum(topic_vector * topic_vector, axis=0, keepdims=True)), EPS_NORM)
    tvn = tvn.astype(jnp.float32)
    w_fused = (rec_vector_2 @ clf_w).astype(jnp.bfloat16)
    bc = clf_b.reshape(1, K).astype(jnp.float32)
    r1 = rec_vector_1.astype(jnp.bfloat16)
    r2 = rec_vector_2.astype(jnp.bfloat16)

    out_shapes = (
        jax.ShapeDtypeStruct((B, K), jnp.float32),
        jax.ShapeDtypeStruct((B, C), jnp.bfloat16),
        jax.ShapeDtypeStruct((B, C), jnp.float32),
        jax.ShapeDtypeStruct((num_tiles * 8, 128), jnp.float32),
    )
    pred, tpn16, tpnn, ae_part = pl.pallas_call(
        _fwd_kernel,
        out_shape=out_shapes,
        grid=(num_tiles,),
        in_specs=[pl.BlockSpec((tile_b, D), lambda i: (i, 0)),
                  pl.BlockSpec((D, C), lambda i: (0, 0)),
                  pl.BlockSpec((C, H), lambda i: (0, 0)),
                  pl.BlockSpec((H, D), lambda i: (0, 0)),
                  pl.BlockSpec((H, K), lambda i: (0, 0)),
                  pl.BlockSpec((1, K), lambda i: (0, 0))],
        out_specs=[pl.BlockSpec((tile_b, K), lambda i: (i, 0)),
                   pl.BlockSpec((tile_b, C), lambda i: (i, 0)),
                   pl.BlockSpec((tile_b, C), lambda i: (i, 0)),
                   pl.BlockSpec((8, 128), lambda i: (i, 0))],
        compiler_params=pltpu.CompilerParams(
            dimension_semantics=("parallel",),
            vmem_limit_bytes=64 * 1024 * 1024),
    )(f_input, tvn, r1, r2, w_fused, bc)

    k_topk = max(B // 4, 1)
    stats = pl.pallas_call(
        functools.partial(_stats_kernel, k_topk, 1.0 / (C * k_topk),
                          1.0 / (1024.0 * B * D), float(C)),
        out_shape=jax.ShapeDtypeStruct((8, 128), jnp.float32),
        grid=(1,),
        in_specs=[pl.BlockSpec((B, C), lambda i: (0, 0)),
                  pl.BlockSpec((num_tiles * 8, 128), lambda i: (0, 0)),
                  pl.BlockSpec((D, C), lambda i: (0, 0))],
        out_specs=pl.BlockSpec((8, 128), lambda i: (0, 0)),
        compiler_params=pltpu.CompilerParams(
            dimension_semantics=("arbitrary",),
            vmem_limit_bytes=64 * 1024 * 1024),
    )(tpn16, ae_part, tvn)

    concept_sim = stats[0, 0]
    ae_loss = stats[1, 0]
    concept_far = stats[2, 0]

    return pred, jnp.float32(0.0), concept_sim, concept_far, tpnn, ae_loss


# speedup vs baseline: 7.7899x; 1.2461x over previous
import functools

import jax
import jax.numpy as jnp
from jax import lax
from jax.experimental import pallas as pl
from jax.experimental.pallas import tpu as pltpu

THRES = 0.3
EPS_NORM = 1e-12


def _prep_kernel(tv_ref, r1_ref, r2_ref, cw_ref,
                 tvn_ref, r1b_ref, r2b_ref, wfb_ref):
    tv = tv_ref[...]
    ss = jnp.sum(tv * tv, axis=0, keepdims=True)
    tvn_ref[...] = tv / jnp.maximum(jnp.sqrt(ss), EPS_NORM)
    r1b_ref[...] = r1_ref[...].astype(jnp.bfloat16)
    r2b = r2_ref[...].astype(jnp.bfloat16)
    r2b_ref[...] = r2b
    wfb_ref[...] = jnp.dot(r2b, cw_ref[...].astype(jnp.bfloat16),
                           preferred_element_type=jnp.float32).astype(jnp.bfloat16)


def _fwd_kernel(f_ref, tvn_ref, r1_ref, r2_ref, wf_ref, bc_ref,
                pred_ref, key_ref, tpnn_ref, ae_ref):
    f = f_ref[...]

    sumsq = jnp.sum(f * f, axis=-1, keepdims=True)
    inv_norm = lax.rsqrt(jnp.maximum(sumsq, EPS_NORM * EPS_NORM))

    topic_prob = jnp.dot(f, tvn_ref[...], preferred_element_type=jnp.float32)
    topic_prob_n = topic_prob * inv_norm

    i32 = pltpu.bitcast(topic_prob_n.astype(jnp.bfloat16).astype(jnp.float32),
                        jnp.int32)
    key_ref[...] = jnp.where(i32 < 0, i32 ^ jnp.int32(0x7FFFFFFF), i32)

    fn = f * inv_norm
    any_hit = jnp.max(topic_prob_n) > THRES

    @pl.when(any_hit)
    def _():
        mask = (topic_prob_n > THRES).astype(jnp.float32)
        topic_prob_am = topic_prob * mask
        topic_prob_sum = jnp.sum(topic_prob_am, axis=-1, keepdims=True) + 0.001
        topic_prob_nn = topic_prob_am * pl.reciprocal(topic_prob_sum, approx=True)

        rec1 = jnp.maximum(
            jnp.dot(topic_prob_nn.astype(jnp.bfloat16), r1_ref[...],
                    preferred_element_type=jnp.float32), 0.0)
        rec1_16 = rec1.astype(jnp.bfloat16)
        rec2 = jnp.dot(rec1_16, r2_ref[...], preferred_element_type=jnp.float32)
        pred = jnp.dot(rec1_16, wf_ref[...], preferred_element_type=jnp.float32) \
            + bc_ref[...]

        diff = fn - rec2
        pred_ref[...] = pred
        tpnn_ref[...] = topic_prob_nn
        ae_ref[...] = jnp.full(ae_ref.shape, jnp.sum(diff * diff), jnp.float32)

    @pl.when(jnp.logical_not(any_hit))
    def _():
        pred_ref[...] = jnp.broadcast_to(bc_ref[...], pred_ref.shape)
        tpnn_ref[...] = jnp.zeros(tpnn_ref.shape, jnp.float32)
        ae_ref[...] = jnp.full(ae_ref.shape, jnp.sum(fn * fn), jnp.float32)


def _code_to_bits(u):
    p = jnp.where(u >= 0x8000, u - 0x8000, 0xFFFF - u)
    return p << 16


def _key32(i):
    return jnp.where(i < 0, i ^ jnp.int32(0x7FFFFFFF), i)


def _stats_kernel(k_topk, inv_ck, ae_scale, c_n,
                  key_ref, ae_ref, tvn_ref, out_ref):
    keys = key_ref[...]

    lo = jnp.full((1, keys.shape[1]), 0x3FFF, jnp.int32)
    hi = jnp.full((1, keys.shape[1]), 0xC000, jnp.int32)

    def body(_, carry):
        lo, hi = carry
        mid = hi - (hi - lo) // 2
        mk = _key32(_code_to_bits(mid))
        cnt = jnp.sum((keys >= mk).astype(jnp.int32), axis=0, keepdims=True)
        ge = cnt >= k_topk
        return jnp.where(ge, mid, lo), jnp.where(ge, hi, mid - 1)

    t_code, _ = lax.fori_loop(0, 16, body, (lo, hi))
    t_bits = _code_to_bits(t_code)
    t_f = pltpu.bitcast(t_bits, jnp.float32)
    t_key = _key32(t_bits)

    gt = keys > t_key
    cnt_gt = jnp.sum(gt.astype(jnp.int32), axis=0, keepdims=True)
    vals = pltpu.bitcast(_key32(keys), jnp.float32)
    sum_gt = jnp.sum(jnp.where(gt, vals, 0.0), axis=0, keepdims=True)
    topk_sum = sum_gt + (k_topk - cnt_gt).astype(jnp.float32) * t_f
    sim_val = -jnp.sum(topk_sum) * inv_ck

    ae_val = jnp.sum(ae_ref[...]) * ae_scale

    s = jnp.sum(tvn_ref[...], axis=1, keepdims=True)
    far_val = (jnp.sum(s * s) - c_n) * (1.0 / (c_n * c_n))

    rows = lax.broadcasted_iota(jnp.int32, (8, 128), 0)
    out_ref[...] = jnp.where(rows == 0, sim_val,
                             jnp.where(rows == 1, ae_val,
                                       jnp.where(rows == 2, far_val, 0.0)))


def kernel(f_input, topic_vector, rec_vector_1, rec_vector_2, clf_w, clf_b):
    B, D = f_input.shape
    C = topic_vector.shape[1]
    H = rec_vector_1.shape[1]
    K = clf_w.shape[1]

    tile_b = 1024 if B % 1024 == 0 else 128
    num_tiles = B // tile_b

    f_input = f_input.astype(jnp.float32)
    bc = clf_b.reshape(1, K).astype(jnp.float32)

    tvn, r1b, r2b, wfb = pl.pallas_call(
        _prep_kernel,
        out_shape=(jax.ShapeDtypeStruct((D, C), jnp.float32),
                   jax.ShapeDtypeStruct((C, H), jnp.bfloat16),
                   jax.ShapeDtypeStruct((H, D), jnp.bfloat16),
                   jax.ShapeDtypeStruct((H, K), jnp.bfloat16)),
    )(topic_vector.astype(jnp.float32), rec_vector_1.astype(jnp.float32),
      rec_vector_2.astype(jnp.float32), clf_w.astype(jnp.float32))

    out_shapes = (
        jax.ShapeDtypeStruct((B, K), jnp.float32),
        jax.ShapeDtypeStruct((B, C), jnp.int32),
        jax.ShapeDtypeStruct((B, C), jnp.float32),
        jax.ShapeDtypeStruct((num_tiles * 8, 128), jnp.float32),
    )
    pred, tpnk, tpnn, ae_part = pl.pallas_call(
        _fwd_kernel,
        out_shape=out_shapes,
        grid=(num_tiles,),
        in_specs=[pl.BlockSpec((tile_b, D), lambda i: (i, 0)),
                  pl.BlockSpec((D, C), lambda i: (0, 0)),
                  pl.BlockSpec((C, H), lambda i: (0, 0)),
                  pl.BlockSpec((H, D), lambda i: (0, 0)),
                  pl.BlockSpec((H, K), lambda i: (0, 0)),
                  pl.BlockSpec((1, K), lambda i: (0, 0))],
        out_specs=[pl.BlockSpec((tile_b, K), lambda i: (i, 0)),
                   pl.BlockSpec((tile_b, C), lambda i: (i, 0)),
                   pl.BlockSpec((tile_b, C), lambda i: (i, 0)),
                   pl.BlockSpec((8, 128), lambda i: (i, 0))],
        compiler_params=pltpu.CompilerParams(
            dimension_semantics=("parallel",),
            vmem_limit_bytes=64 * 1024 * 1024),
    )(f_input, tvn, r1b, r2b, wfb, bc)

    k_topk = max(B // 4, 1)
    stats = pl.pallas_call(
        functools.partial(_stats_kernel, k_topk, 1.0 / (C * k_topk),
                          1.0 / (1024.0 * B * D), float(C)),
        out_shape=jax.ShapeDtypeStruct((8, 128), jnp.float32),
        grid=(1,),
        in_specs=[pl.BlockSpec((B, C), lambda i: (0, 0)),
                  pl.BlockSpec((num_tiles * 8, 128), lambda i: (0, 0)),
                  pl.BlockSpec((D, C), lambda i: (0, 0))],
        out_specs=pl.BlockSpec((8, 128), lambda i: (0, 0)),
        compiler_params=pltpu.CompilerParams(
            dimension_semantics=("arbitrary",),
            vmem_limit_bytes=64 * 1024 * 1024),
    )(tpnk, ae_part, tvn)

    concept_sim = stats[0, 0]
    ae_loss = stats[1, 0]
    concept_far = stats[2, 0]

    return pred, jnp.float32(0.0), concept_sim, concept_far, tpnn, ae_loss
